# baseline (speedup 1.0000x reference)
import functools

import jax
import jax.numpy as jnp
from jax import lax
from jax.experimental import pallas as pl
from jax.experimental.pallas import tpu as pltpu
from jax.experimental.pallas import tpu_sc as plsc

N_NODES = 10000
N_PAD = 10240
NC, NS = 2, 16
NW = NC * NS
CHUNK = 128
GROUP = 4
ROWS_PER_TILE = N_PAD // NS
_BN = (1.0 + 1e-5) ** -0.5


def _sc_mesh():
    return plsc.VectorSubcoreMesh(
        core_axis_name="c", subcore_axis_name="s", num_cores=NC, num_subcores=NS
    )


@functools.lru_cache(maxsize=None)
def _gs_kernel(d, cpt):
    grp = 2 * GROUP if d in (32, 40) else GROUP
    ngroups = cpt // grp
    npairs = ngroups // 2

    def body(t_hbm, src_hbm, dst_hbm, zero_hbm, out_hbm,
             src_v, dst_v, bufs, acc, gsem0, gsem1, ssem0, ssem1):
        cid = lax.axis_index("c")
        sid = lax.axis_index("s")
        wid = cid * NS + sid
        pltpu.sync_copy(src_hbm.at[wid], src_v)
        pltpu.sync_copy(dst_hbm.at[wid], dst_v)
        r0 = sid * ROWS_PER_TILE
        pltpu.sync_copy(zero_hbm.at[pl.ds(r0, ROWS_PER_TILE)],
                        acc.at[pl.ds(r0, ROWS_PER_TILE)])
        plsc.subcore_barrier()

        gsems = (gsem0, gsem1)
        ssems = (ssem0, ssem1)

        def fire_gathers(g, b):
            for j in range(grp):
                pltpu.async_copy(t_hbm.at[src_v.at[g * grp + j]],
                                 bufs.at[b, j], gsems[b])

        def drain_gathers(g, b):
            for j in range(grp):
                pltpu.make_async_copy(t_hbm.at[src_v.at[g * grp + j]],
                                      bufs.at[b, j], gsems[b]).wait()

        def fire_scatters(g, b):
            for j in range(grp):
                pltpu.async_copy(bufs.at[b, j],
                                 acc.at[dst_v.at[g * grp + j]],
                                 ssems[b], add=True)

        def drain_scatters(g, b):
            for j in range(grp):
                pltpu.make_async_copy(bufs.at[b, j],
                                      acc.at[dst_v.at[g * grp + j]],
                                      ssems[b]).wait()

        fire_gathers(0, 0)
        fire_gathers(1, 1)

        def pair(p, carry):
            for b in (0, 1):
                g = p * 2 + b
                drain_gathers(g, b)
                fire_scatters(g, b)
                drain_scatters(g, b)

                @pl.when(g + 2 < ngroups)
                def _():
                    fire_gathers(g + 2, b)
            return carry

        lax.fori_loop(0, npairs, pair, 0)
        plsc.subcore_barrier()
        pltpu.sync_copy(acc.at[pl.ds(r0, ROWS_PER_TILE)],
                        out_hbm.at[cid, pl.ds(r0, ROWS_PER_TILE)])

    return pl.kernel(
        body,
        out_type=jax.ShapeDtypeStruct((NC, N_PAD, d), jnp.float32),
        mesh=_sc_mesh(),
        compiler_params=pltpu.CompilerParams(use_tc_tiling_on_sc=False),
        scratch_types=[
            pltpu.VMEM((cpt, CHUNK), jnp.int32),
            pltpu.VMEM((cpt, CHUNK), jnp.int32),
            pltpu.VMEM((2, grp, CHUNK, d), jnp.float32),
            pltpu.VMEM_SHARED((N_PAD, d), jnp.float32),
            pltpu.SemaphoreType.DMA,
            pltpu.SemaphoreType.DMA,
            pltpu.SemaphoreType.DMA,
            pltpu.SemaphoreType.DMA,
        ],
    )


DEG_W = 8


@functools.lru_cache(maxsize=None)
def _deg_kernel(cpt):
    ngroups = cpt // GROUP
    npairs = ngroups // 2

    def body(dst_hbm, one_hbm, zero_hbm, out_hbm, dst_v, ones_v, acc,
             ssem0, ssem1):
        cid = lax.axis_index("c")
        sid = lax.axis_index("s")
        wid = cid * NS + sid
        pltpu.sync_copy(dst_hbm.at[wid], dst_v)
        pltpu.sync_copy(one_hbm, ones_v)
        r0 = sid * ROWS_PER_TILE
        pltpu.sync_copy(zero_hbm.at[pl.ds(r0, ROWS_PER_TILE)],
                        acc.at[pl.ds(r0, ROWS_PER_TILE)])
        plsc.subcore_barrier()

        ssems = (ssem0, ssem1)

        def fire(g, b):
            for j in range(GROUP):
                pltpu.async_copy(ones_v, acc.at[dst_v.at[g * GROUP + j]],
                                 ssems[b], add=True)

        def drain(g, b):
            for j in range(GROUP):
                pltpu.make_async_copy(ones_v, acc.at[dst_v.at[g * GROUP + j]],
                                      ssems[b]).wait()

        fire(0, 0)
        fire(1, 1)

        def pair(p, carry):
            for b in (0, 1):
                g = p * 2 + b
                drain(g, b)

                @pl.when(g + 2 < ngroups)
                def _():
                    fire(g + 2, b)
            return carry

        lax.fori_loop(0, npairs, pair, 0)
        plsc.subcore_barrier()
        pltpu.sync_copy(acc.at[pl.ds(r0, ROWS_PER_TILE)],
                        out_hbm.at[cid, pl.ds(r0, ROWS_PER_TILE)])

    return pl.kernel(
        body,
        out_type=jax.ShapeDtypeStruct((NC, N_PAD, DEG_W), jnp.float32),
        mesh=_sc_mesh(),
        compiler_params=pltpu.CompilerParams(use_tc_tiling_on_sc=False),
        scratch_types=[
            pltpu.VMEM((cpt, CHUNK), jnp.int32),
            pltpu.VMEM((CHUNK, DEG_W), jnp.float32),
            pltpu.VMEM_SHARED((N_PAD, DEG_W), jnp.float32),
            pltpu.SemaphoreType.DMA,
            pltpu.SemaphoreType.DMA,
        ],
    )


_NB = 1
_R = N_PAD // _NB


def _row_spec(d):
    return pl.BlockSpec((_R, d), lambda i: (i, 0))


def _s_spec(d):
    return pl.BlockSpec((NC, _R, d), lambda i: (0, i, 0))


def _full_spec(shape):
    nd = len(shape)
    return pl.BlockSpec(shape, lambda i: (0,) * nd)


def _elu(u):
    return jnp.where(u > 0, u, jnp.exp(jnp.minimum(u, 0.0)) - 1.0)


def _pad8(d):
    return ((d + 7) // 8) * 8


def _zpad(v, dp):
    d = v.shape[1]
    if dp == d:
        return v
    return jnp.concatenate(
        [v, jnp.zeros((v.shape[0], dp - d), v.dtype)], axis=1)


def _b0_body(x_ref, sdeg_ref, w_ref, dinv_ref, t_ref, h_ref):
    deg = sdeg_ref[0][:, 0:1] + sdeg_ref[1][:, 0:1] + 1.0
    dv = lax.rsqrt(deg)
    h = jnp.dot(x_ref[...], w_ref[...], preferred_element_type=jnp.float32)
    dinv_ref[...] = dv
    h_ref[...] = h
    t_ref[...] = _zpad(h * dv, t_ref.shape[1])


def _tc_b0(x_pad, sdeg, w0):
    din, dout = w0.shape
    dp = _pad8(dout)
    return pl.pallas_call(
        _b0_body,
        grid=(_NB,),
        in_specs=[_row_spec(din), _s_spec(DEG_W), _full_spec((din, dout))],
        out_specs=[_row_spec(1), _row_spec(dp), _row_spec(dout)],
        out_shape=[
            jax.ShapeDtypeStruct((N_PAD, 1), jnp.float32),
            jax.ShapeDtypeStruct((N_PAD, dp), jnp.float32),
            jax.ShapeDtypeStruct((N_PAD, dout), jnp.float32),
        ],
    )(x_pad, sdeg, w0)


def _mid_a_body(s_ref, hp_ref, dinv_ref, bv_ref, ga_ref, be_ref, w_ref,
                t_ref, h_ref):
    dm = hp_ref.shape[1]
    dv = dinv_ref[...]
    u = (dv * (s_ref[0][:, :dm] + s_ref[1][:, :dm])
         + (dv * dv) * hp_ref[...] + bv_ref[...])
    xn = _elu(u) * (ga_ref[...] * _BN) + be_ref[...]
    h = jnp.dot(xn, w_ref[...], preferred_element_type=jnp.float32)
    h_ref[...] = h
    t_ref[...] = _zpad(h * dv, t_ref.shape[1])


def _tc_mid_a(s, h_prev, dinv, bv, ga, be, w):
    dm = h_prev.shape[1]
    dmp = s.shape[2]
    dout = w.shape[1]
    dp = _pad8(dout)
    return pl.pallas_call(
        _mid_a_body,
        grid=(_NB,),
        in_specs=[_s_spec(dmp), _row_spec(dm), _row_spec(1),
                  _full_spec((1, dm)), _full_spec((1, dm)), _full_spec((1, dm)),
                  _full_spec(w.shape)],
        out_specs=[_row_spec(dp), _row_spec(dout)],
        out_shape=[
            jax.ShapeDtypeStruct((N_PAD, dp), jnp.float32),
            jax.ShapeDtypeStruct((N_PAD, dout), jnp.float32),
        ],
    )(s, h_prev, dinv, bv, ga, be, w)


def _a2b_body(s_ref, hp_ref, dinv_ref, bv_ref, ga_ref, be_ref, t_ref, x_ref):
    dm = hp_ref.shape[1]
    dv = dinv_ref[...]
    u = (dv * (s_ref[0][:, :dm] + s_ref[1][:, :dm])
         + (dv * dv) * hp_ref[...] + bv_ref[...])
    xn = _elu(u) * (ga_ref[...] * _BN) + be_ref[...]
    x_ref[...] = xn
    t_ref[...] = _zpad(xn * dv, t_ref.shape[1])


def _tc_a2b(s, h_prev, dinv, bv, ga, be):
    dm = h_prev.shape[1]
    dmp = s.shape[2]
    dp = _pad8(dm)
    return pl.pallas_call(
        _a2b_body,
        grid=(_NB,),
        in_specs=[_s_spec(dmp), _row_spec(dm), _row_spec(1),
                  _full_spec((1, dm)), _full_spec((1, dm)), _full_spec((1, dm))],
        out_specs=[_row_spec(dp), _row_spec(dm)],
        out_shape=[
            jax.ShapeDtypeStruct((N_PAD, dp), jnp.float32),
            jax.ShapeDtypeStruct((N_PAD, dm), jnp.float32),
        ],
    )(s, h_prev, dinv, bv, ga, be)


def _mid_b_body(s_ref, xp_ref, dinv_ref, w_ref, bv_ref, ga_ref, be_ref,
                t_ref, x_ref):
    dm = xp_ref.shape[1]
    dv = dinv_ref[...]
    u = dv * (s_ref[0][:, :dm] + s_ref[1][:, :dm]) + (dv * dv) * xp_ref[...]
    z = jnp.dot(u, w_ref[...], preferred_element_type=jnp.float32) + bv_ref[...]
    xn = _elu(z) * (ga_ref[...] * _BN) + be_ref[...]
    x_ref[...] = xn
    t_ref[...] = _zpad(xn * dv, t_ref.shape[1])


def _tc_mid_b(s, x_prev, dinv, w, bv, ga, be):
    dm = x_prev.shape[1]
    dmp = s.shape[2]
    dout = w.shape[1]
    dp = _pad8(dout)
    return pl.pallas_call(
        _mid_b_body,
        grid=(_NB,),
        in_specs=[_s_spec(dmp), _row_spec(dm), _row_spec(1),
                  _full_spec(w.shape), _full_spec((1, dout)),
                  _full_spec((1, dout)), _full_spec((1, dout))],
        out_specs=[_row_spec(dp), _row_spec(dout)],
        out_shape=[
            jax.ShapeDtypeStruct((N_PAD, dp), jnp.float32),
            jax.ShapeDtypeStruct((N_PAD, dout), jnp.float32),
        ],
    )(s, x_prev, dinv, w, bv, ga, be)


def _final_body(s_ref, xp_ref, dinv_ref, w_ref, bv_ref, o_ref):
    dm = xp_ref.shape[1]
    dv = dinv_ref[...]
    u = dv * (s_ref[0][:, :dm] + s_ref[1][:, :dm]) + (dv * dv) * xp_ref[...]
    o_ref[...] = (jnp.dot(u, w_ref[...], preferred_element_type=jnp.float32)
                  + bv_ref[...])


def _tc_final(s, x_prev, dinv, w, bv):
    dm = x_prev.shape[1]
    dmp = s.shape[2]
    dout = w.shape[1]
    return pl.pallas_call(
        _final_body,
        grid=(_NB,),
        in_specs=[_s_spec(dmp), _row_spec(dm), _row_spec(1),
                  _full_spec(w.shape), _full_spec((1, dout))],
        out_specs=_row_spec(dout),
        out_shape=jax.ShapeDtypeStruct((N_PAD, dout), jnp.float32),
    )(s, x_prev, dinv, w, bv)


def kernel(x, edge_index, Ws, bs, gammas, betas):
    n, e = x.shape[0], edge_index.shape[1]
    cpt = -(-e // (NW * CHUNK))
    cpt = ((cpt + 7) // 8) * 8
    e_pad = NW * cpt * CHUNK
    src = jnp.concatenate(
        [edge_index[0], jnp.zeros((e_pad - e,), jnp.int32)]).reshape(NW, cpt, CHUNK)
    dst = jnp.concatenate(
        [edge_index[1], jnp.full((e_pad - e,), N_NODES, jnp.int32)]).reshape(NW, cpt, CHUNK)
    x_pad = jnp.pad(x, ((0, N_PAD - n), (0, 0)))
    dims = [x.shape[1]] + [w.shape[1] for w in Ws]
    zeros = {dp: jnp.zeros((N_PAD, dp), jnp.float32)
             for dp in set([DEG_W] + [_pad8(d) for d in dims[1:]])}
    bv = [b.reshape(1, -1) for b in bs]
    gv = [g.reshape(1, -1) for g in gammas]
    ev = [b.reshape(1, -1) for b in betas]
    ones_chunk = jnp.ones((CHUNK, DEG_W), jnp.float32)

    def gs(t):
        dp = t.shape[1]
        return _gs_kernel(dp, cpt)(t, src, dst, zeros[dp])

    sdeg = _deg_kernel(cpt)(dst, ones_chunk, zeros[DEG_W])
    dinv, t, h_prev = _tc_b0(x_pad, sdeg, Ws[0])

    s = gs(t)
    for i in range(1, 7):
        t, h_prev = _tc_mid_a(s, h_prev, dinv, bv[i - 1], gv[i - 1], ev[i - 1],
                              Ws[i])
        s = gs(t)
    t, x_prev = _tc_a2b(s, h_prev, dinv, bv[6], gv[6], ev[6])
    s = gs(t)
    for i in range(8, 12):
        t, x_prev = _tc_mid_b(s, x_prev, dinv, Ws[i - 1], bv[i - 1],
                              gv[i - 1], ev[i - 1])
        s = gs(t)
    out = _tc_final(s, x_prev, dinv, Ws[11], bv[11])
    return out[:n]


# speedup vs baseline: 1.0548x; 1.0548x over previous
import functools

import jax
import jax.numpy as jnp
from jax import lax
from jax.experimental import pallas as pl
from jax.experimental.pallas import tpu as pltpu
from jax.experimental.pallas import tpu_sc as plsc

N_NODES = 10000
N_PAD = 10240
NC, NS = 2, 16
NW = NC * NS
CHUNK = 128
GROUP = 4
ROWS_PER_TILE = N_PAD // NS
_BN = (1.0 + 1e-5) ** -0.5


def _sc_mesh():
    return plsc.VectorSubcoreMesh(
        core_axis_name="c", subcore_axis_name="s", num_cores=NC, num_subcores=NS
    )


@functools.lru_cache(maxsize=None)
def _gs_kernel(d, cpt):
    grp = 2 * GROUP if d in (32, 40) else GROUP
    ngroups = cpt // grp
    npairs = ngroups // 2

    def body(t_hbm, src_hbm, dst_hbm, zero_hbm, out_hbm,
             src_v, dst_v, bufs, acc, gsem0, gsem1, ssem0, ssem1):
        cid = lax.axis_index("c")
        sid = lax.axis_index("s")
        wid = cid * NS + sid
        pltpu.sync_copy(src_hbm.at[wid], src_v)
        pltpu.sync_copy(dst_hbm.at[wid], dst_v)
        r0 = sid * ROWS_PER_TILE
        pltpu.sync_copy(zero_hbm.at[pl.ds(r0, ROWS_PER_TILE)],
                        acc.at[pl.ds(r0, ROWS_PER_TILE)])
        plsc.subcore_barrier()

        gsems = (gsem0, gsem1)
        ssems = (ssem0, ssem1)

        def fire_gathers(g, b):
            for j in range(grp):
                pltpu.async_copy(t_hbm.at[src_v.at[g * grp + j]],
                                 bufs.at[b, j], gsems[b])

        def drain_gathers(g, b):
            for j in range(grp):
                pltpu.make_async_copy(t_hbm.at[src_v.at[g * grp + j]],
                                      bufs.at[b, j], gsems[b]).wait()

        def fire_scatters(g, b):
            for j in range(grp):
                pltpu.async_copy(bufs.at[b, j],
                                 acc.at[dst_v.at[g * grp + j]],
                                 ssems[b], add=True)

        def drain_scatters(g, b):
            for j in range(grp):
                pltpu.make_async_copy(bufs.at[b, j],
                                      acc.at[dst_v.at[g * grp + j]],
                                      ssems[b]).wait()

        fire_gathers(0, 0)
        fire_gathers(1, 1)

        def pair(p, carry):
            for b in (0, 1):
                g = p * 2 + b
                drain_gathers(g, b)
                fire_scatters(g, b)
                drain_scatters(g, b)

                @pl.when(g + 2 < ngroups)
                def _():
                    fire_gathers(g + 2, b)
            return carry

        lax.fori_loop(0, npairs, pair, 0)
        plsc.subcore_barrier()
        pltpu.sync_copy(acc.at[pl.ds(r0, ROWS_PER_TILE)],
                        out_hbm.at[cid, pl.ds(r0, ROWS_PER_TILE)])

    return pl.kernel(
        body,
        out_type=jax.ShapeDtypeStruct((NC, N_PAD, d), jnp.float32),
        mesh=_sc_mesh(),
        compiler_params=pltpu.CompilerParams(use_tc_tiling_on_sc=False),
        scratch_types=[
            pltpu.VMEM((cpt, CHUNK), jnp.int32),
            pltpu.VMEM((cpt, CHUNK), jnp.int32),
            pltpu.VMEM((2, grp, CHUNK, d), jnp.float32),
            pltpu.VMEM_SHARED((N_PAD, d), jnp.float32),
            pltpu.SemaphoreType.DMA,
            pltpu.SemaphoreType.DMA,
            pltpu.SemaphoreType.DMA,
            pltpu.SemaphoreType.DMA,
        ],
    )


DEG_W = 8


@functools.lru_cache(maxsize=None)
def _deg_kernel(cpt):
    ngroups = cpt // GROUP
    npairs = ngroups // 2

    def body(dst_hbm, one_hbm, zero_hbm, out_hbm, dst_v, ones_v, acc,
             ssem0, ssem1):
        cid = lax.axis_index("c")
        sid = lax.axis_index("s")
        wid = cid * NS + sid
        pltpu.sync_copy(dst_hbm.at[wid], dst_v)
        pltpu.sync_copy(one_hbm, ones_v)
        r0 = sid * ROWS_PER_TILE
        pltpu.sync_copy(zero_hbm.at[pl.ds(r0, ROWS_PER_TILE)],
                        acc.at[pl.ds(r0, ROWS_PER_TILE)])
        plsc.subcore_barrier()

        ssems = (ssem0, ssem1)

        def fire(g, b):
            for j in range(GROUP):
                pltpu.async_copy(ones_v, acc.at[dst_v.at[g * GROUP + j]],
                                 ssems[b], add=True)

        def drain(g, b):
            for j in range(GROUP):
                pltpu.make_async_copy(ones_v, acc.at[dst_v.at[g * GROUP + j]],
                                      ssems[b]).wait()

        fire(0, 0)
        fire(1, 1)

        def pair(p, carry):
            for b in (0, 1):
                g = p * 2 + b
                drain(g, b)

                @pl.when(g + 2 < ngroups)
                def _():
                    fire(g + 2, b)
            return carry

        lax.fori_loop(0, npairs, pair, 0)
        plsc.subcore_barrier()
        pltpu.sync_copy(acc.at[pl.ds(r0, ROWS_PER_TILE)],
                        out_hbm.at[cid, pl.ds(r0, ROWS_PER_TILE)])

    return pl.kernel(
        body,
        out_type=jax.ShapeDtypeStruct((NC, N_PAD, DEG_W), jnp.float32),
        mesh=_sc_mesh(),
        compiler_params=pltpu.CompilerParams(use_tc_tiling_on_sc=False),
        scratch_types=[
            pltpu.VMEM((cpt, CHUNK), jnp.int32),
            pltpu.VMEM((CHUNK, DEG_W), jnp.float32),
            pltpu.VMEM_SHARED((N_PAD, DEG_W), jnp.float32),
            pltpu.SemaphoreType.DMA,
            pltpu.SemaphoreType.DMA,
        ],
    )


_NB = 2
_R = N_PAD // _NB


def _row_spec(d):
    return pl.BlockSpec((_R, d), lambda i: (i, 0))


def _s_spec(d):
    return pl.BlockSpec((NC, _R, d), lambda i: (0, i, 0))


def _full_spec(shape):
    nd = len(shape)
    return pl.BlockSpec(shape, lambda i: (0,) * nd)


def _elu(u):
    return jnp.where(u > 0, u, jnp.exp(jnp.minimum(u, 0.0)) - 1.0)


def _pad8(d):
    return ((d + 7) // 8) * 8


def _zpad(v, dp):
    d = v.shape[1]
    if dp == d:
        return v
    return jnp.concatenate(
        [v, jnp.zeros((v.shape[0], dp - d), v.dtype)], axis=1)


def _b0_body(x_ref, sdeg_ref, w_ref, dinv_ref, t_ref):
    deg = sdeg_ref[0][:, 0:1] + sdeg_ref[1][:, 0:1] + 1.0
    dv = lax.rsqrt(deg)
    h = jnp.dot(x_ref[...], w_ref[...], preferred_element_type=jnp.float32)
    dinv_ref[...] = dv
    t_ref[...] = _zpad(h * dv, t_ref.shape[1])


def _tc_b0(x_pad, sdeg, w0):
    din, dout = w0.shape
    dp = _pad8(dout)
    return pl.pallas_call(
        _b0_body,
        grid=(_NB,),
        in_specs=[_row_spec(din), _s_spec(DEG_W), _full_spec((din, dout))],
        out_specs=[_row_spec(1), _row_spec(dp)],
        out_shape=[
            jax.ShapeDtypeStruct((N_PAD, 1), jnp.float32),
            jax.ShapeDtypeStruct((N_PAD, dp), jnp.float32),
        ],
    )(x_pad, sdeg, w0)


def _mid_a_body(s_ref, tp_ref, dinv_ref, bv_ref, ga_ref, be_ref, w_ref,
                t_ref):
    dm = bv_ref.shape[1]
    dv = dinv_ref[...]
    u = (dv * (s_ref[0][:, :dm] + s_ref[1][:, :dm] + tp_ref[:, :dm])
         + bv_ref[...])
    xn = _elu(u) * (ga_ref[...] * _BN) + be_ref[...]
    h = jnp.dot(xn, w_ref[...], preferred_element_type=jnp.float32)
    t_ref[...] = _zpad(h * dv, t_ref.shape[1])


def _tc_mid_a(s, t_prev, dinv, bv, ga, be, w):
    dmp = s.shape[2]
    dout = w.shape[1]
    dp = _pad8(dout)
    dm = bv.shape[1]
    return pl.pallas_call(
        _mid_a_body,
        grid=(_NB,),
        in_specs=[_s_spec(dmp), _row_spec(dmp), _row_spec(1),
                  _full_spec((1, dm)), _full_spec((1, dm)), _full_spec((1, dm)),
                  _full_spec(w.shape)],
        out_specs=_row_spec(dp),
        out_shape=jax.ShapeDtypeStruct((N_PAD, dp), jnp.float32),
    )(s, t_prev, dinv, bv, ga, be, w)


def _a2b_body(s_ref, tp_ref, dinv_ref, bv_ref, ga_ref, be_ref, t_ref):
    dm = bv_ref.shape[1]
    dv = dinv_ref[...]
    u = (dv * (s_ref[0][:, :dm] + s_ref[1][:, :dm] + tp_ref[:, :dm])
         + bv_ref[...])
    xn = _elu(u) * (ga_ref[...] * _BN) + be_ref[...]
    t_ref[...] = _zpad(xn * dv, t_ref.shape[1])


def _tc_a2b(s, t_prev, dinv, bv, ga, be):
    dmp = s.shape[2]
    dm = bv.shape[1]
    dp = _pad8(dm)
    return pl.pallas_call(
        _a2b_body,
        grid=(_NB,),
        in_specs=[_s_spec(dmp), _row_spec(dmp), _row_spec(1),
                  _full_spec((1, dm)), _full_spec((1, dm)), _full_spec((1, dm))],
        out_specs=_row_spec(dp),
        out_shape=jax.ShapeDtypeStruct((N_PAD, dp), jnp.float32),
    )(s, t_prev, dinv, bv, ga, be)


def _mid_b_body(s_ref, tp_ref, dinv_ref, w_ref, bv_ref, ga_ref, be_ref,
                t_ref):
    dm = w_ref.shape[0]
    dv = dinv_ref[...]
    u = dv * (s_ref[0][:, :dm] + s_ref[1][:, :dm] + tp_ref[:, :dm])
    z = jnp.dot(u, w_ref[...], preferred_element_type=jnp.float32) + bv_ref[...]
    xn = _elu(z) * (ga_ref[...] * _BN) + be_ref[...]
    t_ref[...] = _zpad(xn * dv, t_ref.shape[1])


def _tc_mid_b(s, t_prev, dinv, w, bv, ga, be):
    dmp = s.shape[2]
    dout = w.shape[1]
    dp = _pad8(dout)
    return pl.pallas_call(
        _mid_b_body,
        grid=(_NB,),
        in_specs=[_s_spec(dmp), _row_spec(dmp), _row_spec(1),
                  _full_spec(w.shape), _full_spec((1, dout)),
                  _full_spec((1, dout)), _full_spec((1, dout))],
        out_specs=_row_spec(dp),
        out_shape=jax.ShapeDtypeStruct((N_PAD, dp), jnp.float32),
    )(s, t_prev, dinv, w, bv, ga, be)


def _final_body(s_ref, tp_ref, dinv_ref, w_ref, bv_ref, o_ref):
    dm = w_ref.shape[0]
    dv = dinv_ref[...]
    u = dv * (s_ref[0][:, :dm] + s_ref[1][:, :dm] + tp_ref[:, :dm])
    o_ref[...] = (jnp.dot(u, w_ref[...], preferred_element_type=jnp.float32)
                  + bv_ref[...])


def _tc_final(s, t_prev, dinv, w, bv):
    dmp = s.shape[2]
    dout = w.shape[1]
    return pl.pallas_call(
        _final_body,
        grid=(_NB,),
        in_specs=[_s_spec(dmp), _row_spec(dmp), _row_spec(1),
                  _full_spec(w.shape), _full_spec((1, dout))],
        out_specs=_row_spec(dout),
        out_shape=jax.ShapeDtypeStruct((N_PAD, dout), jnp.float32),
    )(s, t_prev, dinv, w, bv)


def kernel(x, edge_index, Ws, bs, gammas, betas):
    n, e = x.shape[0], edge_index.shape[1]
    cpt = -(-e // (NW * CHUNK))
    cpt = ((cpt + 7) // 8) * 8
    e_pad = NW * cpt * CHUNK
    src = jnp.concatenate(
        [edge_index[0], jnp.zeros((e_pad - e,), jnp.int32)]).reshape(NW, cpt, CHUNK)
    dst = jnp.concatenate(
        [edge_index[1], jnp.full((e_pad - e,), N_NODES, jnp.int32)]).reshape(NW, cpt, CHUNK)
    x_pad = jnp.pad(x, ((0, N_PAD - n), (0, 0)))
    dims = [x.shape[1]] + [w.shape[1] for w in Ws]
    zeros = {dp: jnp.zeros((N_PAD, dp), jnp.float32)
             for dp in set([DEG_W] + [_pad8(d) for d in dims[1:]])}
    bv = [b.reshape(1, -1) for b in bs]
    gv = [g.reshape(1, -1) for g in gammas]
    ev = [b.reshape(1, -1) for b in betas]
    ones_chunk = jnp.ones((CHUNK, DEG_W), jnp.float32)

    def gs(t):
        dp = t.shape[1]
        return _gs_kernel(dp, cpt)(t, src, dst, zeros[dp])

    sdeg = _deg_kernel(cpt)(dst, ones_chunk, zeros[DEG_W])
    dinv, t = _tc_b0(x_pad, sdeg, Ws[0])

    s = gs(t)
    for i in range(1, 7):
        t = _tc_mid_a(s, t, dinv, bv[i - 1], gv[i - 1], ev[i - 1], Ws[i])
        s = gs(t)
    t = _tc_a2b(s, t, dinv, bv[6], gv[6], ev[6])
    s = gs(t)
    for i in range(8, 12):
        t = _tc_mid_b(s, t, dinv, Ws[i - 1], bv[i - 1], gv[i - 1], ev[i - 1])
        s = gs(t)
    out = _tc_final(s, t, dinv, Ws[11], bv[11])
    return out[:n]
